# baseline (speedup 1.0000x reference)
import functools

import jax
import jax.numpy as jnp
from jax import lax
from jax.experimental import pallas as pl
from jax.experimental.pallas import tpu as pltpu
from jax.experimental.pallas import tpu_sc as plsc

GROUPS = 8
K = 32
EPS = 1e-5
RB = 2048
NW = 32


def _elu(x):
    return jnp.where(x > 0, x, jnp.exp(x) - 1.0)


def _gn_affine(s, q, gamma, beta, gsize, count):
    c = s.shape[0]
    rg = lax.broadcasted_iota(jnp.int32, (c, c), 0) // gsize
    cg = lax.broadcasted_iota(jnp.int32, (c, c), 1) // gsize
    gmat = jnp.where(rg == cg, jnp.float32(1.0 / count), jnp.float32(0.0))
    dn = (((1,), (0,)), ((), ()))
    mean = lax.dot_general(gmat, s, dn, preferred_element_type=jnp.float32)
    ex2 = lax.dot_general(gmat, q, dn, preferred_element_type=jnp.float32)
    var = ex2 - mean * mean
    sc = gamma * lax.rsqrt(var + EPS)
    sh = beta - mean * sc
    return sc, sh


def _topk_body(xtb_ref, xp_ref, idx_ref, d2_s):
    n = xp_ref.shape[0]
    nb = n // RB
    dn = (((1,), (0,)), ((), ()))
    f32 = jnp.float32

    xp = xp_ref[...]
    sqc = jnp.sum(xp * xp, axis=1, keepdims=True)
    io = lax.broadcasted_iota(jnp.int32, (n, RB), 0)
    inf = f32(jnp.inf)

    def _argmin(d2v):
        return jnp.argmin(d2v, axis=0).reshape(1, RB)

    def _block(rb, _):
        xrb = xtb_ref[rb]
        sqr = jnp.sum(xrb * xrb, axis=0, keepdims=True)
        d = lax.dot_general(xp, xrb, dn, preferred_element_type=f32)
        d2_s[...] = sqc + sqr - 2.0 * d

        d2v = d2_s[...]
        j = _argmin(d2v)
        d2_s[...] = jnp.where(io == j, inf, d2v)

        def _it(k, _c):
            d2v = d2_s[...]
            j = _argmin(d2v)
            d2_s[...] = jnp.where(io == j, inf, d2v)
            idx_ref[k, rb] = j + bidx * n
            return 0

        lax.fori_loop(0, K, _it, 0)
        return 0

    lax.fori_loop(0, nb, _block, 0)


def _conv_body(xt_ref, nbrs_ref, a1_ref, wd1_ref, b1_ref, g1_ref, be1_ref,
               w11_ref, b11_ref, g11_ref, be11_ref,
               w20_ref, b20_ref, g20_ref, be20_ref,
               w21_ref, b21_ref, g21_ref, be21_ref,
               out_ref, pooled_s):
    n = xt_ref.shape[1]
    dn = (((1,), (0,)), ((), ()))
    f32 = jnp.float32
    inf = f32(jnp.inf)

    p1 = lax.dot_general(a1_ref[...], xt_ref[...], dn,
                         preferred_element_type=f32)
    p1 = p1 + b1_ref[...]
    wd1 = wd1_ref[...]

    def _bb(k, carry):
        s1, q1 = carry
        f1 = p1 + lax.dot_general(wd1, nbrs_ref[k], dn,
                                  preferred_element_type=f32)
        return (s1 + jnp.sum(f1, axis=1, keepdims=True),
                q1 + jnp.sum(f1 * f1, axis=1, keepdims=True))

    s1, q1 = lax.fori_loop(0, K, _bb, (jnp.zeros((32, 1), f32),
                                       jnp.zeros((32, 1), f32)))
    sc1, sh1 = _gn_affine(s1, q1, g1_ref[...], be1_ref[...], 4, 4 * n * K)

    pooled_s[...] = jnp.full((64, n), -inf, f32)
    w11 = w11_ref[...]
    b11 = b11_ref[...]

    def _bd(k, carry):
        s2, q2 = carry
        f1 = p1 + lax.dot_general(wd1, nbrs_ref[k], dn,
                                  preferred_element_type=f32)
        e1 = _elu(f1 * sc1 + sh1)
        f2 = lax.dot_general(w11, e1, dn, preferred_element_type=f32) + b11
        pooled_s[...] = jnp.maximum(pooled_s[...], f2)
        return (s2 + jnp.sum(f2, axis=1, keepdims=True),
                q2 + jnp.sum(f2 * f2, axis=1, keepdims=True))

    s2, q2 = lax.fori_loop(0, K, _bd, (jnp.zeros((64, 1), f32),
                                       jnp.zeros((64, 1), f32)))
    sc2, sh2 = _gn_affine(s2, q2, g11_ref[...], be11_ref[...], 8, 8 * n * K)

    h2 = _elu(pooled_s[...] * sc2 + sh2)

    f3 = lax.dot_general(w20_ref[...], h2, dn,
                         preferred_element_type=f32) + b20_ref[...]
    s3 = jnp.sum(f3, axis=1, keepdims=True)
    q3 = jnp.sum(f3 * f3, axis=1, keepdims=True)
    sc3, sh3 = _gn_affine(s3, q3, g20_ref[...], be20_ref[...], 16, 16 * n)
    h3 = _elu(f3 * sc3 + sh3)

    f4 = lax.dot_general(w21_ref[...], h3, dn,
                         preferred_element_type=f32) + b21_ref[...]
    s4 = jnp.sum(f4, axis=1, keepdims=True)
    q4 = jnp.sum(f4 * f4, axis=1, keepdims=True)
    sc4, sh4 = _gn_affine(s4, q4, g21_ref[...], be21_ref[...], 32, 32 * n)
    out_ref[...] = _elu(f4 * sc4 + sh4)


def _sc_gather(table, idx):
    e = idx.shape[0]
    per_w = e // NW
    ch = min(4096, per_w)
    nch = per_w // ch
    mesh = plsc.VectorSubcoreMesh(core_axis_name="c", subcore_axis_name="s")

    @functools.partial(
        pl.kernel, mesh=mesh,
        out_type=jax.ShapeDtypeStruct((e, 16), jnp.float32),
        scratch_types=[
            pltpu.VMEM((ch,), jnp.int32),
            pltpu.VMEM((ch, 16), jnp.float32),
            pltpu.SemaphoreType.DMA,
        ],
        compiler_params=pltpu.CompilerParams(use_tc_tiling_on_sc=False),
    )
    def _k(table_hbm, idx_hbm, out_hbm, idx_v, rows_v, sem):
        wid = lax.axis_index("s") * 2 + lax.axis_index("c")
        for c in range(nch):
            base = wid * per_w + c * ch
            pltpu.sync_copy(idx_hbm.at[pl.ds(base, ch)], idx_v)
            pltpu.async_copy(table_hbm.at[idx_v], rows_v, sem).wait()
            pltpu.sync_copy(rows_v, out_hbm.at[pl.ds(base, ch)])

    return _k(table, idx)


@jax.jit
def kernel(points, W1_0, b1_0, g1_0, be1_0, W1_1, b1_1, g1_1, be1_1,
           W2_0, b2_0, g2_0, be2_0, W2_1, b2_1, g2_1, be2_1):
    f32 = jnp.float32
    b, _, n = points.shape
    nb = n // RB
    xt = jnp.concatenate([points, jnp.zeros((b, 5, n), f32)], axis=1)
    xp = jnp.transpose(xt, (0, 2, 1))
    xtb = jnp.transpose(xt.reshape(b, 8, nb, RB), (0, 2, 1, 3))

    wp, wd = W1_0[:, :3], W1_0[:, 3:]
    a1 = jnp.pad(wp - wd, ((0, 0), (0, 5)))
    wd1 = jnp.pad(wd, ((0, 0), (0, 13)))
    col = lambda v: v.reshape(-1, 1)
    wspec = lambda shp: pl.BlockSpec(shp, lambda i: (0, 0))

    topk_call = pl.pallas_call(
        _topk_body,
        grid=(1,),
        in_specs=[
            pl.BlockSpec((nb, 8, RB), lambda i: (0, 0, 0)),
            pl.BlockSpec((n, 8), lambda i: (0, 0)),
        ],
        out_specs=pl.BlockSpec((K, nb, 1, RB), lambda i: (0, 0, 0, 0)),
        out_shape=jax.ShapeDtypeStruct((K, nb, 1, RB), jnp.int32),
        scratch_shapes=[pltpu.VMEM((n, RB), f32)],
        compiler_params=pltpu.CompilerParams(
            vmem_limit_bytes=100 * 1024 * 1024),
    )

    conv_call = pl.pallas_call(
        _conv_body,
        grid=(1,),
        in_specs=[
            pl.BlockSpec((8, n), lambda i: (0, 0)),
            pl.BlockSpec((K, 16, n), lambda i: (0, 0, 0)),
            wspec((32, 8)), wspec((32, 16)),
            wspec((32, 1)), wspec((32, 1)), wspec((32, 1)),
            wspec((64, 32)),
            wspec((64, 1)), wspec((64, 1)), wspec((64, 1)),
            wspec((128, 64)),
            wspec((128, 1)), wspec((128, 1)), wspec((128, 1)),
            wspec((256, 128)),
            wspec((256, 1)), wspec((256, 1)), wspec((256, 1)),
        ],
        out_specs=pl.BlockSpec((256, n), lambda i: (0, 0)),
        out_shape=jax.ShapeDtypeStruct((256, n), f32),
        scratch_shapes=[pltpu.VMEM((64, n), f32)],
        compiler_params=pltpu.CompilerParams(
            vmem_limit_bytes=100 * 1024 * 1024),
    )

    outs = []
    for bi in range(b):
        idx_b = topk_call(xtb[bi], xp[bi])
        table_b = jnp.pad(xp[bi], ((0, 0), (0, 8)))
        rows_b = _sc_gather(table_b, idx_b.reshape(K * n))
        nbrs_b = jnp.transpose(rows_b.reshape(K, n, 16), (0, 2, 1))
        outs.append(conv_call(
            xt[bi], nbrs_b, a1, wd1,
            col(b1_0), col(g1_0), col(be1_0),
            W1_1, col(b1_1), col(g1_1), col(be1_1),
            W2_0, col(b2_0), col(g2_0), col(be2_0),
            W2_1, col(b2_1), col(g2_1), col(be2_1)))
    return jnp.stack(outs, axis=0)


# speedup vs baseline: 1.1439x; 1.1439x over previous
import functools

import jax
import jax.numpy as jnp
from jax import lax
from jax.experimental import pallas as pl
from jax.experimental.pallas import tpu as pltpu
from jax.experimental.pallas import tpu_sc as plsc

GROUPS = 8
K = 32
EPS = 1e-5
RB = 2048
NW = 32


def _elu(x):
    return jnp.where(x > 0, x, jnp.exp(x) - 1.0)


def _gn_affine(s, q, gamma, beta, gsize, count):
    c = s.shape[0]
    rg = lax.broadcasted_iota(jnp.int32, (c, c), 0) // gsize
    cg = lax.broadcasted_iota(jnp.int32, (c, c), 1) // gsize
    gmat = jnp.where(rg == cg, jnp.float32(1.0 / count), jnp.float32(0.0))
    dn = (((1,), (0,)), ((), ()))
    mean = lax.dot_general(gmat, s, dn, preferred_element_type=jnp.float32)
    ex2 = lax.dot_general(gmat, q, dn, preferred_element_type=jnp.float32)
    var = ex2 - mean * mean
    sc = gamma * lax.rsqrt(var + EPS)
    sh = beta - mean * sc
    return sc, sh


def _topk_body(xtb_ref, xp_ref, idx_ref, d2_s):
    n = xp_ref.shape[0]
    nb = n // RB
    dn = (((1,), (0,)), ((), ()))
    f32 = jnp.float32

    bidx = pl.program_id(0)
    xp = xp_ref[...]
    sqc = jnp.sum(xp * xp, axis=1, keepdims=True)
    io = lax.broadcasted_iota(jnp.int32, (n, RB), 0)
    inf = f32(jnp.inf)

    def _argmin(d2v):
        return jnp.argmin(d2v, axis=0).reshape(1, RB)

    def _block(rb, _):
        xrb = xtb_ref[rb]
        sqr = jnp.sum(xrb * xrb, axis=0, keepdims=True)
        d = lax.dot_general(xp, xrb, dn, preferred_element_type=f32)
        d2_s[...] = sqc + sqr - 2.0 * d

        d2v = d2_s[...]
        j = _argmin(d2v)
        d2_s[...] = jnp.where(io == j, inf, d2v)

        def _it(k, _c):
            d2v = d2_s[...]
            j = _argmin(d2v)
            d2_s[...] = jnp.where(io == j, inf, d2v)
            idx_ref[k, rb] = j + bidx * n
            return 0

        lax.fori_loop(0, K, _it, 0)
        return 0

    lax.fori_loop(0, nb, _block, 0)


def _conv_body(xt_ref, nbrs_ref, a1_ref, wd1_ref, b1_ref, g1_ref, be1_ref,
               w11_ref, b11_ref, g11_ref, be11_ref,
               w20_ref, b20_ref, g20_ref, be20_ref,
               w21_ref, b21_ref, g21_ref, be21_ref,
               out_ref, pooled_s, f1_s):
    n = xt_ref.shape[1]
    dn = (((1,), (0,)), ((), ()))
    f32 = jnp.float32
    inf = f32(jnp.inf)

    p1 = lax.dot_general(a1_ref[...], xt_ref[...], dn,
                         preferred_element_type=f32)
    p1 = p1 + b1_ref[...]
    wd1 = wd1_ref[...]

    def _bb(k, carry):
        s1, q1 = carry
        f1 = p1 + lax.dot_general(wd1, nbrs_ref[k], dn,
                                  preferred_element_type=f32)
        f1_s[k] = f1
        return (s1 + jnp.sum(f1, axis=1, keepdims=True),
                q1 + jnp.sum(f1 * f1, axis=1, keepdims=True))

    s1, q1 = lax.fori_loop(0, K, _bb, (jnp.zeros((32, 1), f32),
                                       jnp.zeros((32, 1), f32)))
    sc1, sh1 = _gn_affine(s1, q1, g1_ref[...], be1_ref[...], 4, 4 * n * K)

    pooled_s[...] = jnp.full((64, n), -inf, f32)
    w11 = w11_ref[...]
    b11 = b11_ref[...]

    def _bd(k, carry):
        s2, q2 = carry
        e1 = _elu(f1_s[k] * sc1 + sh1)
        f2 = lax.dot_general(w11, e1, dn, preferred_element_type=f32) + b11
        pooled_s[...] = jnp.maximum(pooled_s[...], f2)
        return (s2 + jnp.sum(f2, axis=1, keepdims=True),
                q2 + jnp.sum(f2 * f2, axis=1, keepdims=True))

    s2, q2 = lax.fori_loop(0, K, _bd, (jnp.zeros((64, 1), f32),
                                       jnp.zeros((64, 1), f32)))
    sc2, sh2 = _gn_affine(s2, q2, g11_ref[...], be11_ref[...], 8, 8 * n * K)

    h2 = _elu(pooled_s[...] * sc2 + sh2)

    f3 = lax.dot_general(w20_ref[...], h2, dn,
                         preferred_element_type=f32) + b20_ref[...]
    s3 = jnp.sum(f3, axis=1, keepdims=True)
    q3 = jnp.sum(f3 * f3, axis=1, keepdims=True)
    sc3, sh3 = _gn_affine(s3, q3, g20_ref[...], be20_ref[...], 16, 16 * n)
    h3 = _elu(f3 * sc3 + sh3)

    f4 = lax.dot_general(w21_ref[...], h3, dn,
                         preferred_element_type=f32) + b21_ref[...]
    s4 = jnp.sum(f4, axis=1, keepdims=True)
    q4 = jnp.sum(f4 * f4, axis=1, keepdims=True)
    sc4, sh4 = _gn_affine(s4, q4, g21_ref[...], be21_ref[...], 32, 32 * n)
    out_ref[...] = _elu(f4 * sc4 + sh4)


def _sc_gather(table, idx):
    e = idx.shape[0]
    per_w = e // NW
    ch = 4096
    nch = per_w // ch
    mesh = plsc.VectorSubcoreMesh(core_axis_name="c", subcore_axis_name="s")

    @functools.partial(
        pl.kernel, mesh=mesh,
        out_type=jax.ShapeDtypeStruct((e, 16), jnp.float32),
        scratch_types=[
            pltpu.VMEM((ch,), jnp.int32),
            pltpu.VMEM((ch, 16), jnp.float32),
            pltpu.SemaphoreType.DMA,
        ],
        compiler_params=pltpu.CompilerParams(use_tc_tiling_on_sc=False),
    )
    def _k(table_hbm, idx_hbm, out_hbm, idx_v, rows_v, sem):
        wid = lax.axis_index("s") * 2 + lax.axis_index("c")
        for c in range(nch):
            base = wid * per_w + c * ch
            pltpu.sync_copy(idx_hbm.at[pl.ds(base, ch)], idx_v)
            pltpu.async_copy(table_hbm.at[idx_v], rows_v, sem).wait()
            pltpu.sync_copy(rows_v, out_hbm.at[pl.ds(base, ch)])

    return _k(table, idx)


@jax.jit
def kernel(points, W1_0, b1_0, g1_0, be1_0, W1_1, b1_1, g1_1, be1_1,
           W2_0, b2_0, g2_0, be2_0, W2_1, b2_1, g2_1, be2_1):
    f32 = jnp.float32
    b, _, n = points.shape
    nb = n // RB
    xt = jnp.concatenate([points, jnp.zeros((b, 5, n), f32)], axis=1)
    xp = jnp.transpose(xt, (0, 2, 1))
    xtb = jnp.transpose(xt.reshape(b, 8, nb, RB), (0, 2, 1, 3))

    idx = pl.pallas_call(
        _topk_body,
        grid=(b,),
        in_specs=[
            pl.BlockSpec((None, nb, 8, RB), lambda i: (i, 0, 0, 0)),
            pl.BlockSpec((None, n, 8), lambda i: (i, 0, 0)),
        ],
        out_specs=pl.BlockSpec((None, K, nb, 1, RB), lambda i: (i, 0, 0, 0, 0)),
        out_shape=jax.ShapeDtypeStruct((b, K, nb, 1, RB), jnp.int32),
        scratch_shapes=[pltpu.VMEM((n, RB), f32)],
        compiler_params=pltpu.CompilerParams(
            vmem_limit_bytes=100 * 1024 * 1024),
    )(xtb, xp)

    table = jnp.pad(xp, ((0, 0), (0, 0), (0, 8))).reshape(b * n, 16)
    nbr_rows = _sc_gather(table, idx.reshape(b * K * n))
    nbrs = jnp.transpose(nbr_rows.reshape(b, K, n, 16), (0, 1, 3, 2))

    wp, wd = W1_0[:, :3], W1_0[:, 3:]
    a1 = jnp.pad(wp - wd, ((0, 0), (0, 5)))
    wd1 = jnp.pad(wd, ((0, 0), (0, 13)))
    col = lambda v: v.reshape(-1, 1)
    wspec = lambda shp: pl.BlockSpec(shp, lambda i: (0, 0))

    out = pl.pallas_call(
        _conv_body,
        grid=(b,),
        in_specs=[
            pl.BlockSpec((None, 8, n), lambda i: (i, 0, 0)),
            pl.BlockSpec((None, K, 16, n), lambda i: (i, 0, 0, 0)),
            wspec((32, 8)), wspec((32, 16)),
            wspec((32, 1)), wspec((32, 1)), wspec((32, 1)),
            wspec((64, 32)),
            wspec((64, 1)), wspec((64, 1)), wspec((64, 1)),
            wspec((128, 64)),
            wspec((128, 1)), wspec((128, 1)), wspec((128, 1)),
            wspec((256, 128)),
            wspec((256, 1)), wspec((256, 1)), wspec((256, 1)),
        ],
        out_specs=pl.BlockSpec((None, 256, n), lambda i: (i, 0, 0)),
        out_shape=jax.ShapeDtypeStruct((b, 256, n), f32),
        scratch_shapes=[pltpu.VMEM((64, n), f32),
                        pltpu.VMEM((K, 32, n), f32)],
        compiler_params=pltpu.CompilerParams(
            vmem_limit_bytes=100 * 1024 * 1024),
    )(xt, nbrs, a1, wd1,
      col(b1_0), col(g1_0), col(be1_0),
      W1_1, col(b1_1), col(g1_1), col(be1_1),
      W2_0, col(b2_0), col(g2_0), col(be2_0),
      W2_1, col(b2_1), col(g2_1), col(be2_1))
    return out
